# baseline (speedup 1.0000x reference)
import functools

import jax
import jax.numpy as jnp
from jax import lax
from jax.experimental import pallas as pl
from jax.experimental.pallas import tpu as pltpu
from jax.experimental.pallas import tpu_sc as plsc

N = 10000
E = 320000
R = 8
D_IN = 128
H = 16
C_OUT = 16

NC = 2
NS = 16
NW = NC * NS
CHUNK = 128
CH = 2 * (-(-E // (NW * CHUNK * 2)))
E_PAD = NW * CH * CHUNK

S_ROWS = N * R + 16
S_TRASH = N * R


def _sc_mesh():
    return plsc.VectorSubcoreMesh(
        core_axis_name="c", subcore_axis_name="s", num_cores=NC, num_subcores=NS
    )


SEGS = 8
SEGR = CH * CHUNK // SEGS


def _edge_pass(table, gidx, sidx, ones, zeros, with_counts):
    stripe = S_ROWS // NS
    out_t = jax.ShapeDtypeStruct((NC, S_ROWS, 16), jnp.float32)

    @functools.partial(
        pl.kernel,
        out_type=(out_t, out_t) if with_counts else out_t,
        mesh=_sc_mesh(),
        scratch_types=[
            pltpu.VMEM_SHARED((S_ROWS, 16), jnp.float32),
            [pltpu.VMEM((SEGR,), jnp.int32) for _ in range(2)],
            [pltpu.VMEM((SEGR,), jnp.int32) for _ in range(2)],
            [pltpu.VMEM((SEGR, 16), jnp.float32) for _ in range(2)],
            pltpu.SemaphoreType.DMA,
            [pltpu.SemaphoreType.DMA for _ in range(2)],
        ],
        compiler_params=pltpu.CompilerParams(use_tc_tiling_on_sc=False),
    )
    def kern(table_hbm, gidx_hbm, sidx_hbm, ones_hbm, zeros_hbm, *refs):
        if with_counts:
            cout_hbm, sout_hbm, acc, gidx_v, sidx_v, rows_v, gsem, ssem = refs
        else:
            sout_hbm, acc, gidx_v, sidx_v, rows_v, gsem, ssem = refs
        c = lax.axis_index("c")
        s = lax.axis_index("s")
        wid = c * NS + s
        row0 = s * stripe

        def zero_acc():
            pltpu.sync_copy(zeros_hbm.at[pl.ds(row0, stripe)],
                            acc.at[pl.ds(row0, stripe)])

        def readout(dst_hbm):
            pltpu.sync_copy(acc.at[pl.ds(row0, stripe)],
                            dst_hbm.at[c, pl.ds(row0, stripe)])

        zero_acc()
        if with_counts:
            pltpu.sync_copy(ones_hbm, rows_v[0])
        plsc.subcore_barrier()

        if with_counts:
            for seg in range(SEGS):
                pltpu.sync_copy(sidx_hbm.at[wid, seg], sidx_v[seg % 2])
                pltpu.sync_copy(rows_v[0], acc.at[sidx_v[seg % 2]], add=True)
            plsc.subcore_barrier()
            readout(cout_hbm)
            plsc.subcore_barrier()
            zero_acc()
            plsc.subcore_barrier()

        scat = [None, None]
        for seg in range(SEGS):
            b = seg % 2
            if scat[b] is not None:
                scat[b].wait()
            pltpu.sync_copy(gidx_hbm.at[wid, seg], gidx_v[b])
            pltpu.sync_copy(sidx_hbm.at[wid, seg], sidx_v[b])
            pltpu.async_copy(table_hbm.at[gidx_v[b]], rows_v[b], gsem).wait()
            scat[b] = pltpu.async_copy(rows_v[b], acc.at[sidx_v[b]], ssem[b],
                                       add=True)
        scat[0].wait()
        scat[1].wait()
        plsc.subcore_barrier()
        readout(sout_hbm)

    return kern(table, gidx, sidx, ones, zeros)


def _prep_kernel(x_ref, wcat_ref, root_ref, b_ref, xw_ref, base_ref):
    x = x_ref[...]
    xw_ref[...] = jnp.dot(x, wcat_ref[...], preferred_element_type=jnp.float32)
    base_ref[...] = (
        jnp.dot(x, root_ref[...], preferred_element_type=jnp.float32)
        + b_ref[...]
    )


def _prep(x, wcat, root, b, d_in, blk):
    grid = N // blk
    return pl.pallas_call(
        _prep_kernel,
        grid=(grid,),
        in_specs=[
            pl.BlockSpec((blk, d_in), lambda i: (i, 0)),
            pl.BlockSpec((d_in, R * H), lambda i: (0, 0)),
            pl.BlockSpec((d_in, H), lambda i: (0, 0)),
            pl.BlockSpec((1, H), lambda i: (0, 0)),
        ],
        out_specs=[
            pl.BlockSpec((blk, R * H), lambda i: (i, 0)),
            pl.BlockSpec((blk, H), lambda i: (i, 0)),
        ],
        out_shape=[
            jax.ShapeDtypeStruct((N, R * H), jnp.float32),
            jax.ShapeDtypeStruct((N, H), jnp.float32),
        ],
    )(x, wcat, root, b)


def _agg_message(sa_ref, sb_ref, ca_ref, cb_ref):
    svals = sa_ref[...] + sb_ref[...]
    cnt = ca_ref[...] + cb_ref[...]
    msg = jnp.zeros(sa_ref.shape[:1] + (16,), jnp.float32)
    for r in range(R):
        inv = 1.0 / jnp.maximum(cnt[:, r * 16 : r * 16 + 1], 1.0)
        msg = msg + svals[:, r * 16 : (r + 1) * 16] * inv
    return msg


def _mid_kernel(sa_ref, sb_ref, ca_ref, cb_ref, base_ref, wcat_ref,
                root_ref, b_ref, hw_ref, base2_ref):
    h = jax.nn.relu(base_ref[...] + _agg_message(sa_ref, sb_ref, ca_ref, cb_ref))
    hw_ref[...] = jnp.dot(h, wcat_ref[...], preferred_element_type=jnp.float32)
    base2_ref[...] = (
        jnp.dot(h, root_ref[...], preferred_element_type=jnp.float32)
        + b_ref[...]
    )


def _final_kernel(sa_ref, sb_ref, ca_ref, cb_ref, base_ref, out_ref):
    out_ref[...] = base_ref[...] + _agg_message(sa_ref, sb_ref, ca_ref, cb_ref)


def kernel(x, edge_index, edge_type, W1, root1, b1, W2, root2, b2):
    src = edge_index[0].astype(jnp.int32)
    dst = edge_index[1].astype(jnp.int32)
    et = edge_type.astype(jnp.int32)

    gidx = src * R + et
    sidx = dst * R + et
    pad = E_PAD - E
    gidx = jnp.concatenate([gidx, jnp.zeros((pad,), jnp.int32)])
    sidx = jnp.concatenate([sidx, jnp.full((pad,), S_TRASH, jnp.int32)])
    gidx = gidx.reshape(NW, SEGS, SEGR)
    sidx = sidx.reshape(NW, SEGS, SEGR)

    wcat1 = W1.transpose(1, 0, 2).reshape(D_IN, R * H).astype(jnp.float32)
    wcat2 = W2.transpose(1, 0, 2).reshape(H, R * C_OUT).astype(jnp.float32)

    zeros_s = jnp.zeros((S_ROWS, 16), jnp.float32)
    ones_c = jnp.ones((SEGR, 16), jnp.float32)

    xw1, base1 = _prep(x, wcat1, root1.astype(jnp.float32),
                       b1.reshape(1, H).astype(jnp.float32), D_IN, 1000)
    cacc, s1 = _edge_pass(xw1.reshape(N * R, 16), gidx, sidx, ones_c,
                          zeros_s, with_counts=True)
    ca = cacc[0, : N * R, :].reshape(N, R * 16)
    cb = cacc[1, : N * R, :].reshape(N, R * 16)
    s1a = s1[0, : N * R, :].reshape(N, R * 16)
    s1b = s1[1, : N * R, :].reshape(N, R * 16)

    blk = 1000
    full = lambda shape: pl.BlockSpec(shape, lambda i: (0, 0))
    rowblk = lambda w: pl.BlockSpec((blk, w), lambda i: (i, 0))

    hw2, base2 = pl.pallas_call(
        _mid_kernel,
        grid=(N // blk,),
        in_specs=[
            rowblk(R * 16), rowblk(R * 16), rowblk(R * 16), rowblk(R * 16),
            rowblk(H), full((H, R * C_OUT)), full((H, C_OUT)), full((1, C_OUT)),
        ],
        out_specs=[rowblk(R * C_OUT), rowblk(C_OUT)],
        out_shape=[
            jax.ShapeDtypeStruct((N, R * C_OUT), jnp.float32),
            jax.ShapeDtypeStruct((N, C_OUT), jnp.float32),
        ],
    )(s1a, s1b, ca, cb, base1, wcat2, root2.astype(jnp.float32),
      b2.reshape(1, C_OUT).astype(jnp.float32))

    s2 = _edge_pass(hw2.reshape(N * R, 16), gidx, sidx, ones_c, zeros_s,
                    with_counts=False)
    s2a = s2[0, : N * R, :].reshape(N, R * 16)
    s2b = s2[1, : N * R, :].reshape(N, R * 16)

    out = pl.pallas_call(
        _final_kernel,
        grid=(N // blk,),
        in_specs=[rowblk(R * 16), rowblk(R * 16), rowblk(R * 16),
                  rowblk(R * 16), rowblk(C_OUT)],
        out_specs=rowblk(C_OUT),
        out_shape=jax.ShapeDtypeStruct((N, C_OUT), jnp.float32),
    )(s2a, s2b, ca, cb, base2)

    return out


# speedup vs baseline: 1.5160x; 1.5160x over previous
import functools

import jax
import jax.numpy as jnp
from jax import lax
from jax.experimental import pallas as pl
from jax.experimental.pallas import tpu as pltpu
from jax.experimental.pallas import tpu_sc as plsc

N = 10000
E = 320000
R = 8
D_IN = 128
H = 16
C_OUT = 16

NC = 2
NS = 16
NW = NC * NS
CHUNK = 128
CH = 2 * (-(-E // (NW * CHUNK * 2)))
E_PAD = NW * CH * CHUNK

S_ROWS = N * R + 16
S_TRASH = N * R


def _sc_mesh():
    return plsc.VectorSubcoreMesh(
        core_axis_name="c", subcore_axis_name="s", num_cores=NC, num_subcores=NS
    )


SEGS = 8
SEGR = CH * CHUNK // SEGS


def _edge_pass(table, gidx, sidx, ones, zeros, with_counts):
    stripe = S_ROWS // NS
    out_t = jax.ShapeDtypeStruct((NC, S_ROWS, 16), jnp.float32)

    @functools.partial(
        pl.kernel,
        out_type=(out_t, out_t) if with_counts else out_t,
        mesh=_sc_mesh(),
        scratch_types=[
            pltpu.VMEM_SHARED((S_ROWS, 16), jnp.float32),
            [pltpu.VMEM((SEGR,), jnp.int32) for _ in range(2)],
            [pltpu.VMEM((SEGR,), jnp.int32) for _ in range(2)],
            [pltpu.VMEM((SEGR, 16), jnp.float32) for _ in range(2)],
            pltpu.SemaphoreType.DMA,
            [pltpu.SemaphoreType.DMA for _ in range(2)],
        ],
        compiler_params=pltpu.CompilerParams(use_tc_tiling_on_sc=False),
    )
    def kern(table_hbm, gidx_hbm, sidx_hbm, ones_hbm, zeros_hbm, *refs):
        if with_counts:
            cout_hbm, sout_hbm, acc, gidx_v, sidx_v, rows_v, gsem, ssem = refs
        else:
            sout_hbm, acc, gidx_v, sidx_v, rows_v, gsem, ssem = refs
        c = lax.axis_index("c")
        s = lax.axis_index("s")
        wid = c * NS + s
        row0 = s * stripe

        def zero_acc():
            pltpu.sync_copy(zeros_hbm.at[pl.ds(row0, stripe)],
                            acc.at[pl.ds(row0, stripe)])

        def readout(dst_hbm):
            pltpu.sync_copy(acc.at[pl.ds(row0, stripe)],
                            dst_hbm.at[c, pl.ds(row0, stripe)])

        zero_acc()
        if with_counts:
            pltpu.sync_copy(ones_hbm, rows_v[0])
        plsc.subcore_barrier()

        if with_counts:
            for seg in range(SEGS):
                pltpu.sync_copy(sidx_hbm.at[wid, seg], sidx_v[seg % 2])
                pltpu.sync_copy(rows_v[0], acc.at[sidx_v[seg % 2]], add=True)
            plsc.subcore_barrier()
            readout(cout_hbm)
            plsc.subcore_barrier()
            zero_acc()
            plsc.subcore_barrier()

        scat = [None, None]
        for seg in range(SEGS):
            b = seg % 2
            if scat[b] is not None:
                scat[b].wait()
            pltpu.sync_copy(gidx_hbm.at[wid, seg], gidx_v[b])
            pltpu.sync_copy(sidx_hbm.at[wid, seg], sidx_v[b])
            pltpu.async_copy(table_hbm.at[gidx_v[b]], rows_v[b], gsem).wait()
            scat[b] = pltpu.async_copy(rows_v[b], acc.at[sidx_v[b]], ssem[b],
                                       add=True)
        scat[0].wait()
        scat[1].wait()
        plsc.subcore_barrier()
        readout(sout_hbm)

    return kern(table, gidx, sidx, ones, zeros)


def _prep_kernel(x_ref, wcat_ref, root_ref, b_ref, xw_ref, base_ref):
    x = x_ref[...]
    xw_ref[...] = jnp.dot(x, wcat_ref[...], preferred_element_type=jnp.float32)
    base_ref[...] = (
        jnp.dot(x, root_ref[...], preferred_element_type=jnp.float32)
        + b_ref[...]
    )


def _prep(x, wcat, root, b, d_in, blk):
    grid = N // blk
    return pl.pallas_call(
        _prep_kernel,
        grid=(grid,),
        in_specs=[
            pl.BlockSpec((blk, d_in), lambda i: (i, 0)),
            pl.BlockSpec((d_in, R * H), lambda i: (0, 0)),
            pl.BlockSpec((d_in, H), lambda i: (0, 0)),
            pl.BlockSpec((1, H), lambda i: (0, 0)),
        ],
        out_specs=[
            pl.BlockSpec((blk, R * H), lambda i: (i, 0)),
            pl.BlockSpec((blk, H), lambda i: (i, 0)),
        ],
        out_shape=[
            jax.ShapeDtypeStruct((N, R * H), jnp.float32),
            jax.ShapeDtypeStruct((N, H), jnp.float32),
        ],
    )(x, wcat, root, b)


def _agg_message(s_ref, c_ref, blk):
    svals = s_ref[0] + s_ref[1]
    cnt = c_ref[0] + c_ref[1]
    scaled = svals * (1.0 / jnp.maximum(cnt, 1.0))
    return scaled.reshape(blk, R, 16).sum(axis=1)


def _mid_kernel(s_ref, c_ref, base_ref, wcat_ref, root_ref, b_ref,
                hw_ref, base2_ref):
    blk = base_ref.shape[0]
    h = jax.nn.relu(base_ref[...] + _agg_message(s_ref, c_ref, blk))
    hw_ref[...] = jnp.dot(h, wcat_ref[...], preferred_element_type=jnp.float32)
    base2_ref[...] = (
        jnp.dot(h, root_ref[...], preferred_element_type=jnp.float32)
        + b_ref[...]
    )


def _final_kernel(s_ref, c_ref, base_ref, out_ref):
    out_ref[...] = base_ref[...] + _agg_message(s_ref, c_ref, base_ref.shape[0])


def kernel(x, edge_index, edge_type, W1, root1, b1, W2, root2, b2):
    src = edge_index[0].astype(jnp.int32)
    dst = edge_index[1].astype(jnp.int32)
    et = edge_type.astype(jnp.int32)

    gidx = src * R + et
    sidx = dst * R + et
    pad = E_PAD - E
    gidx = jnp.concatenate([gidx, jnp.zeros((pad,), jnp.int32)])
    sidx = jnp.concatenate([sidx, jnp.full((pad,), S_TRASH, jnp.int32)])
    gidx = gidx.reshape(NW, SEGS, SEGR)
    sidx = sidx.reshape(NW, SEGS, SEGR)

    wcat1 = W1.transpose(1, 0, 2).reshape(D_IN, R * H).astype(jnp.float32)
    wcat2 = W2.transpose(1, 0, 2).reshape(H, R * C_OUT).astype(jnp.float32)

    zeros_s = jnp.zeros((S_ROWS, 16), jnp.float32)
    ones_c = jnp.ones((SEGR, 16), jnp.float32)

    xw1, base1 = _prep(x, wcat1, root1.astype(jnp.float32),
                       b1.reshape(1, H).astype(jnp.float32), D_IN, 1000)
    cacc, s1 = _edge_pass(xw1.reshape(N * R, 16), gidx, sidx, ones_c,
                          zeros_s, with_counts=True)

    blk = 1000
    full = lambda shape: pl.BlockSpec(shape, lambda i: (0, 0))
    rowblk = lambda w: pl.BlockSpec((blk, w), lambda i: (i, 0))
    accblk = pl.BlockSpec((NC, blk * R, 16), lambda i: (0, i, 0))

    hw2, base2 = pl.pallas_call(
        _mid_kernel,
        grid=(N // blk,),
        in_specs=[
            accblk, accblk,
            rowblk(H), full((H, R * C_OUT)), full((H, C_OUT)), full((1, C_OUT)),
        ],
        out_specs=[rowblk(R * C_OUT), rowblk(C_OUT)],
        out_shape=[
            jax.ShapeDtypeStruct((N, R * C_OUT), jnp.float32),
            jax.ShapeDtypeStruct((N, C_OUT), jnp.float32),
        ],
    )(s1, cacc, base1, wcat2, root2.astype(jnp.float32),
      b2.reshape(1, C_OUT).astype(jnp.float32))

    s2 = _edge_pass(hw2.reshape(N * R, 16), gidx, sidx, ones_c, zeros_s,
                    with_counts=False)

    out = pl.pallas_call(
        _final_kernel,
        grid=(N // blk,),
        in_specs=[accblk, accblk, rowblk(C_OUT)],
        out_specs=rowblk(C_OUT),
        out_shape=jax.ShapeDtypeStruct((N, C_OUT), jnp.float32),
    )(s2, cacc, base2)

    return out


# speedup vs baseline: 2.4935x; 1.6448x over previous
import functools

import jax
import jax.numpy as jnp
from jax import lax
from jax.experimental import pallas as pl
from jax.experimental.pallas import tpu as pltpu
from jax.experimental.pallas import tpu_sc as plsc

N = 10000
E = 320000
R = 8
D_IN = 128
H = 16
C_OUT = 16

NC = 2
NS = 16
NW = NC * NS
CHUNK = 128
CH = 2 * (-(-E // (NW * CHUNK * 2)))
E_PAD = NW * CH * CHUNK

S_ROWS = N * R + 16
S_TRASH = N * R


def _sc_mesh():
    return plsc.VectorSubcoreMesh(
        core_axis_name="c", subcore_axis_name="s", num_cores=NC, num_subcores=NS
    )


SEGS = 8
SEGR = CH * CHUNK // SEGS


def _edge_pass(table, gidx, sidx, ones, zeros, with_counts):
    stripe = S_ROWS // NS
    out_t = jax.ShapeDtypeStruct((NC, S_ROWS, 16), jnp.float32)

    @functools.partial(
        pl.kernel,
        out_type=(out_t, out_t) if with_counts else out_t,
        mesh=_sc_mesh(),
        scratch_types=[
            pltpu.VMEM_SHARED((S_ROWS, 16), jnp.float32),
            [pltpu.VMEM((SEGR,), jnp.int32) for _ in range(2)],
            [pltpu.VMEM((SEGR,), jnp.int32) for _ in range(2)],
            [pltpu.VMEM((SEGR, 16), jnp.float32) for _ in range(2)],
            pltpu.SemaphoreType.DMA,
            [pltpu.SemaphoreType.DMA for _ in range(2)],
        ],
        compiler_params=pltpu.CompilerParams(use_tc_tiling_on_sc=False),
    )
    def kern(table_hbm, gidx_hbm, sidx_hbm, ones_hbm, zeros_hbm, *refs):
        if with_counts:
            cout_hbm, sout_hbm, acc, gidx_v, sidx_v, rows_v, gsem, ssem = refs
        else:
            sout_hbm, acc, gidx_v, sidx_v, rows_v, gsem, ssem = refs
        c = lax.axis_index("c")
        s = lax.axis_index("s")
        wid = c * NS + s
        row0 = s * stripe

        def zero_acc():
            pltpu.sync_copy(zeros_hbm.at[pl.ds(row0, stripe)],
                            acc.at[pl.ds(row0, stripe)])

        def readout(dst_hbm):
            pltpu.sync_copy(acc.at[pl.ds(row0, stripe)],
                            dst_hbm.at[c, pl.ds(row0, stripe)])

        zero_acc()
        if with_counts:
            pltpu.sync_copy(ones_hbm, rows_v[0])
        plsc.subcore_barrier()

        if with_counts:
            for seg in range(SEGS):
                pltpu.sync_copy(sidx_hbm.at[wid, seg], sidx_v[seg % 2])
                pltpu.sync_copy(rows_v[0], acc.at[sidx_v[seg % 2]], add=True)
            plsc.subcore_barrier()
            readout(cout_hbm)
            plsc.subcore_barrier()
            zero_acc()
            plsc.subcore_barrier()

        scat = [None, None]
        for seg in range(SEGS):
            b = seg % 2
            if scat[b] is not None:
                scat[b].wait()
            pltpu.sync_copy(gidx_hbm.at[wid, seg], gidx_v[b])
            pltpu.sync_copy(sidx_hbm.at[wid, seg], sidx_v[b])
            pltpu.async_copy(table_hbm.at[gidx_v[b]], rows_v[b], gsem).wait()
            scat[b] = pltpu.async_copy(rows_v[b], acc.at[sidx_v[b]], ssem[b],
                                       add=True)
        scat[0].wait()
        scat[1].wait()
        plsc.subcore_barrier()
        readout(sout_hbm)

    return kern(table, gidx, sidx, ones, zeros)


def _prep_kernel(x_ref, wcat_ref, root_ref, b_ref, xw_ref, base_ref):
    x = x_ref[...]
    xw_ref[...] = jnp.dot(x, wcat_ref[...], preferred_element_type=jnp.float32)
    base_ref[...] = (
        jnp.dot(x, root_ref[...], preferred_element_type=jnp.float32)
        + b_ref[...]
    )


def _prep(x, wcat, root, b, d_in, blk):
    grid = N // blk
    return pl.pallas_call(
        _prep_kernel,
        grid=(grid,),
        in_specs=[
            pl.BlockSpec((blk, d_in), lambda i: (i, 0)),
            pl.BlockSpec((d_in, R * H), lambda i: (0, 0)),
            pl.BlockSpec((d_in, H), lambda i: (0, 0)),
            pl.BlockSpec((1, H), lambda i: (0, 0)),
        ],
        out_specs=[
            pl.BlockSpec((blk, R * H), lambda i: (i, 0)),
            pl.BlockSpec((blk, H), lambda i: (i, 0)),
        ],
        out_shape=[
            jax.ShapeDtypeStruct((N, R * H), jnp.float32),
            jax.ShapeDtypeStruct((N, H), jnp.float32),
        ],
    )(x, wcat, root, b)


def _agg_message(s_ref, c_ref, blk):
    svals = s_ref[0] + s_ref[1]
    cnt = c_ref[0] + c_ref[1]
    scaled = svals * (1.0 / jnp.maximum(cnt, 1.0))
    msg = scaled[:, 0:16]
    for r in range(1, R):
        msg = msg + scaled[:, r * 16 : (r + 1) * 16]
    return msg


def _mid_kernel(s_ref, c_ref, base_ref, wcat_ref, root_ref, b_ref,
                hw_ref, base2_ref):
    blk = base_ref.shape[0]
    h = jax.nn.relu(base_ref[...] + _agg_message(s_ref, c_ref, blk))
    hw_ref[...] = jnp.dot(h, wcat_ref[...], preferred_element_type=jnp.float32)
    base2_ref[...] = (
        jnp.dot(h, root_ref[...], preferred_element_type=jnp.float32)
        + b_ref[...]
    )


def _final_kernel(s_ref, c_ref, base_ref, out_ref):
    out_ref[...] = base_ref[...] + _agg_message(s_ref, c_ref, base_ref.shape[0])


def kernel(x, edge_index, edge_type, W1, root1, b1, W2, root2, b2):
    src = edge_index[0].astype(jnp.int32)
    dst = edge_index[1].astype(jnp.int32)
    et = edge_type.astype(jnp.int32)

    gidx = src * R + et
    sidx = dst * R + et
    pad = E_PAD - E
    gidx = jnp.concatenate([gidx, jnp.zeros((pad,), jnp.int32)])
    sidx = jnp.concatenate([sidx, jnp.full((pad,), S_TRASH, jnp.int32)])
    gidx = gidx.reshape(NW, SEGS, SEGR)
    sidx = sidx.reshape(NW, SEGS, SEGR)

    wcat1 = W1.transpose(1, 0, 2).reshape(D_IN, R * H).astype(jnp.float32)
    wcat2 = W2.transpose(1, 0, 2).reshape(H, R * C_OUT).astype(jnp.float32)

    zeros_s = jnp.zeros((S_ROWS, 16), jnp.float32)
    ones_c = jnp.ones((SEGR, 16), jnp.float32)

    xw1, base1 = _prep(x, wcat1, root1.astype(jnp.float32),
                       b1.reshape(1, H).astype(jnp.float32), D_IN, 1000)
    cacc, s1 = _edge_pass(xw1.reshape(N * R, 16), gidx, sidx, ones_c,
                          zeros_s, with_counts=True)

    blk = 1000
    full = lambda shape: pl.BlockSpec(shape, lambda i: (0, 0))
    rowblk = lambda w: pl.BlockSpec((blk, w), lambda i: (i, 0))
    accblk = pl.BlockSpec((NC, blk, 128), lambda i: (0, i, 0))
    s1v = s1.reshape(NC, S_ROWS * 16 // 128, 128)
    cav = cacc.reshape(NC, S_ROWS * 16 // 128, 128)

    hw2, base2 = pl.pallas_call(
        _mid_kernel,
        grid=(N // blk,),
        in_specs=[
            accblk, accblk,
            rowblk(H), full((H, R * C_OUT)), full((H, C_OUT)), full((1, C_OUT)),
        ],
        out_specs=[rowblk(R * C_OUT), rowblk(C_OUT)],
        out_shape=[
            jax.ShapeDtypeStruct((N, R * C_OUT), jnp.float32),
            jax.ShapeDtypeStruct((N, C_OUT), jnp.float32),
        ],
    )(s1v, cav, base1, wcat2, root2.astype(jnp.float32),
      b2.reshape(1, C_OUT).astype(jnp.float32))

    s2 = _edge_pass(hw2.reshape(N * R, 16), gidx, sidx, ones_c, zeros_s,
                    with_counts=False)

    out = pl.pallas_call(
        _final_kernel,
        grid=(N // blk,),
        in_specs=[accblk, accblk, rowblk(C_OUT)],
        out_specs=rowblk(C_OUT),
        out_shape=jax.ShapeDtypeStruct((N, C_OUT), jnp.float32),
    )(s2.reshape(NC, S_ROWS * 16 // 128, 128), cav, base2)

    return out
